# baseline (speedup 1.0000x reference)
import jax
import jax.numpy as jnp
from jax.experimental import pallas as pl

_NORM = 100.0
_THRESH2 = 100.0
_PREC = jax.lax.Precision.HIGHEST


def _silu(x):
    return x * jax.nn.sigmoid(x)


def _dot(a, b):
    return jax.lax.dot_general(
        a, b, (((a.ndim - 1,), (0,)), ((), ())),
        preferred_element_type=jnp.float32, precision=_PREC)


def _xattn_body(hl_ref, xl_ref, hp_ref, xp_ref, lm_ref, pmt_ref,
                Wa1l_ref, Wa1p_ref, wa1d_ref, ba1_ref,
                Wa2_ref, ba2_ref, wa3_ref, ba3_ref,
                Wv1p_ref, wv1d_ref, bv1_ref, Wv2_ref, bv2_ref,
                Wc1l_ref, Wc1p_ref, wc1d_ref, bc1_ref, wc2_ref, bc2_ref,
                h_out_ref, x_out_ref):
    hl = hl_ref[0]
    xl = xl_ref[0]
    hp = hp_ref[0]
    xp = xp_ref[0]
    lm = lm_ref[0]
    pmt = pmt_ref[0]
    nl = hl.shape[0]
    hid = Wa2_ref.shape[0]

    xl2 = jnp.sum(xl * xl, axis=1, keepdims=True)
    xp2 = jnp.sum(xp * xp, axis=1, keepdims=True)
    d2 = jnp.maximum(xl2 + xp2.T - 2.0 * _dot(xl, xp.T), 0.0)
    dist = jnp.sqrt(d2 + 1e-8)
    em = (d2 < _THRESH2).astype(jnp.float32)

    hlA = _dot(hl, Wa1l_ref[:])
    hpA = _dot(hp, Wa1p_ref[:])
    hlC = _dot(hl, Wc1l_ref[:])
    hpC = _dot(hp, Wc1p_ref[:])
    hpV = _dot(hp, Wv1p_ref[:])

    d2e = d2[:, :, None]

    a1 = _silu(hlA[:, None, :] + hpA[None, :, :]
               + d2e * wa1d_ref[:][None] + ba1_ref[:][None])
    a2 = _silu(_dot(a1.reshape(-1, hid), Wa2_ref[:]) + ba2_ref[:])
    a2 = a2.reshape(a1.shape)
    att = jax.nn.sigmoid(jnp.sum(a2 * wa3_ref[:][None], axis=2)
                         + ba3_ref[0, 0])
    ae = att * pmt * em

    v1 = _silu(hpV[None, :, :] + d2e * wv1d_ref[:][None] + bv1_ref[:][None])
    t = jnp.sum(ae[:, :, None] * v1, axis=1)
    s = jnp.sum(ae, axis=1, keepdims=True)
    h_cross = (_dot(t, Wv2_ref[:]) + s * bv2_ref[:]) * (1.0 / _NORM) * lm

    c1 = _silu(hlC[:, None, :] + hpC[None, :, :]
               + d2e * wc1d_ref[:][None] + bc1_ref[:][None])
    c = jnp.tanh(jnp.sum(c1 * wc2_ref[:][None], axis=2) + bc2_ref[0, 0])
    g = c * pmt * em / (dist + 1e-8)
    gs = jnp.sum(g, axis=1, keepdims=True)
    x_cross = (xl * gs - _dot(g, xp)) * (1.0 / _NORM) * lm

    h_out_ref[0] = h_cross
    x_out_ref[0] = x_cross


@jax.jit
def kernel(h_ligand, x_ligand, h_protein, x_protein, ligand_mask, protein_mask,
           W_a1, b_a1, W_a2, b_a2, W_a3, b_a3,
           W_v1, b_v1, W_v2, b_v2, W_c1, b_c1, W_c2, b_c2):
    bs, nl, lig_nf = h_ligand.shape
    npn, prot_nf = h_protein.shape[1], h_protein.shape[2]
    hid = W_a2.shape[0]
    f32 = jnp.float32
    TL = 16
    nt = nl // TL

    pmt = jnp.transpose(protein_mask, (0, 2, 1))

    args = (
        h_ligand, x_ligand, h_protein, x_protein, ligand_mask, pmt,
        W_a1[:lig_nf], W_a1[lig_nf:lig_nf + prot_nf], W_a1[lig_nf + prot_nf:],
        b_a1.reshape(1, -1),
        W_a2, b_a2.reshape(1, -1), W_a3.T, b_a3.reshape(1, 1),
        W_v1[:prot_nf], W_v1[prot_nf:], b_v1.reshape(1, -1),
        W_v2, b_v2.reshape(1, -1),
        W_c1[:lig_nf], W_c1[lig_nf:lig_nf + prot_nf], W_c1[lig_nf + prot_nf:],
        b_c1.reshape(1, -1), W_c2.T, b_c2.reshape(1, 1),
    )

    def lig_spec(a):
        shp = a.shape
        return pl.BlockSpec((1, TL) + shp[2:],
                            lambda b, t: (b, t) + (0,) * (len(shp) - 2))

    def batch_spec(a):
        shp = a.shape
        return pl.BlockSpec((1,) + shp[1:],
                            lambda b, t: (b,) + (0,) * (len(shp) - 1))

    def full_spec(a):
        shp = a.shape
        return pl.BlockSpec(shp, lambda b, t: (0,) * len(shp))

    in_specs = ([lig_spec(args[0]), lig_spec(args[1]),
                 batch_spec(args[2]), batch_spec(args[3]),
                 lig_spec(args[4]), batch_spec(args[5])]
                + [full_spec(a) for a in args[6:]])

    h_cross, x_cross = pl.pallas_call(
        _xattn_body,
        grid=(bs, nt),
        in_specs=in_specs,
        out_specs=[
            pl.BlockSpec((1, TL, lig_nf), lambda b, t: (b, t, 0)),
            pl.BlockSpec((1, TL, 3), lambda b, t: (b, t, 0)),
        ],
        out_shape=[
            jax.ShapeDtypeStruct((bs, nl, lig_nf), f32),
            jax.ShapeDtypeStruct((bs, nl, 3), f32),
        ],
    )(*args)
    return (h_cross, x_cross)


# speedup vs baseline: 1.6155x; 1.6155x over previous
import jax
import jax.numpy as jnp
from jax.experimental import pallas as pl
from jax.experimental.pallas import tpu as pltpu

_NORM = 100.0
_THRESH2 = 100.0


def _silu(x):
    return x * jax.nn.sigmoid(x)


def _dot(a, b, prec=jax.lax.Precision.DEFAULT):
    return jax.lax.dot_general(
        a, b, (((a.ndim - 1,), (0,)), ((), ())),
        preferred_element_type=jnp.float32, precision=prec)


def _xattn_body(hl_ref, xl_ref, hp_ref, xp_ref, lm_ref, pmt_ref,
                Wa1l_ref, Wa1p_ref, wa1d_ref, ba1_ref,
                Wa2_ref, ba2_ref, wa3_ref, ba3_ref,
                Wv1p_ref, wv1d_ref, bv1_ref, Wv2_ref, bv2_ref,
                Wc1l_ref, Wc1p_ref, wc1d_ref, bc1_ref, wc2_ref, bc2_ref,
                h_out_ref, x_out_ref,
                hpA_ref, hpC_ref, hpV_ref):
    hl = hl_ref[0]
    xl = xl_ref[0]
    hp = hp_ref[0]
    xp = xp_ref[0]
    lm = lm_ref[0]
    pmt = pmt_ref[0]

    @pl.when(pl.program_id(1) == 0)
    def _():
        hpA_ref[:] = _dot(hp, Wa1p_ref[:])
        hpC_ref[:] = _dot(hp, Wc1p_ref[:])
        hpV_ref[:] = _dot(hp, Wv1p_ref[:]) + bv1_ref[:]

    xl2 = jnp.sum(xl * xl, axis=1, keepdims=True)
    xp2 = jnp.sum(xp * xp, axis=1, keepdims=True)
    d2 = jnp.maximum(xl2 + xp2.T - 2.0 * _dot(xl, xp.T, jax.lax.Precision.HIGHEST),
                     0.0)
    dist = jnp.sqrt(d2 + 1e-8)
    em = (d2 < _THRESH2).astype(jnp.float32)

    hlA = _dot(hl, Wa1l_ref[:]) + ba1_ref[:]
    hlC = _dot(hl, Wc1l_ref[:]) + bc1_ref[:]

    d2e = d2[:, :, None]

    a1 = _silu(hlA[:, None, :] + hpA_ref[:][None] + d2e * wa1d_ref[:][None])
    a2 = _silu(_dot(a1, Wa2_ref[:]) + ba2_ref[:][None])
    att = jax.nn.sigmoid(jnp.sum(a2 * wa3_ref[:][None], axis=2)
                         + ba3_ref[0, 0])
    ae = att * pmt * em

    v1 = _silu(hpV_ref[:][None] + d2e * wv1d_ref[:][None])
    t = jnp.sum(ae[:, :, None] * v1, axis=1)
    s = jnp.sum(ae, axis=1, keepdims=True)
    h_cross = (_dot(t, Wv2_ref[:]) + s * bv2_ref[:]) * (1.0 / _NORM) * lm

    c1 = _silu(hlC[:, None, :] + hpC_ref[:][None] + d2e * wc1d_ref[:][None])
    c = jnp.tanh(jnp.sum(c1 * wc2_ref[:][None], axis=2) + bc2_ref[0, 0])
    g = c * pmt * em / (dist + 1e-8)
    gs = jnp.sum(g, axis=1, keepdims=True)
    x_cross = (xl * gs - _dot(g, xp)) * (1.0 / _NORM) * lm

    h_out_ref[0] = h_cross
    x_out_ref[0] = x_cross


@jax.jit
def kernel(h_ligand, x_ligand, h_protein, x_protein, ligand_mask, protein_mask,
           W_a1, b_a1, W_a2, b_a2, W_a3, b_a3,
           W_v1, b_v1, W_v2, b_v2, W_c1, b_c1, W_c2, b_c2):
    bs, nl, lig_nf = h_ligand.shape
    npn, prot_nf = h_protein.shape[1], h_protein.shape[2]
    hid = W_a2.shape[0]
    f32 = jnp.float32
    TL = 24
    nt = nl // TL

    pmt = jnp.transpose(protein_mask, (0, 2, 1))

    args = (
        h_ligand, x_ligand, h_protein, x_protein, ligand_mask, pmt,
        W_a1[:lig_nf], W_a1[lig_nf:lig_nf + prot_nf], W_a1[lig_nf + prot_nf:],
        b_a1.reshape(1, -1),
        W_a2, b_a2.reshape(1, -1), W_a3.T, b_a3.reshape(1, 1),
        W_v1[:prot_nf], W_v1[prot_nf:], b_v1.reshape(1, -1),
        W_v2, b_v2.reshape(1, -1),
        W_c1[:lig_nf], W_c1[lig_nf:lig_nf + prot_nf], W_c1[lig_nf + prot_nf:],
        b_c1.reshape(1, -1), W_c2.T, b_c2.reshape(1, 1),
    )

    def lig_spec(a):
        shp = a.shape
        return pl.BlockSpec((1, TL) + shp[2:],
                            lambda b, t: (b, t) + (0,) * (len(shp) - 2))

    def batch_spec(a):
        shp = a.shape
        return pl.BlockSpec((1,) + shp[1:],
                            lambda b, t: (b,) + (0,) * (len(shp) - 1))

    def full_spec(a):
        shp = a.shape
        return pl.BlockSpec(shp, lambda b, t: (0,) * len(shp))

    in_specs = ([lig_spec(args[0]), lig_spec(args[1]),
                 batch_spec(args[2]), batch_spec(args[3]),
                 lig_spec(args[4]), batch_spec(args[5])]
                + [full_spec(a) for a in args[6:]])

    h_cross, x_cross = pl.pallas_call(
        _xattn_body,
        grid=(bs, nt),
        in_specs=in_specs,
        out_specs=[
            pl.BlockSpec((1, TL, lig_nf), lambda b, t: (b, t, 0)),
            pl.BlockSpec((1, TL, 3), lambda b, t: (b, t, 0)),
        ],
        out_shape=[
            jax.ShapeDtypeStruct((bs, nl, lig_nf), f32),
            jax.ShapeDtypeStruct((bs, nl, 3), f32),
        ],
        scratch_shapes=[
            pltpu.VMEM((npn, hid), f32),
            pltpu.VMEM((npn, hid), f32),
            pltpu.VMEM((npn, hid), f32),
        ],
    )(*args)
    return (h_cross, x_cross)


# speedup vs baseline: 1.7756x; 1.0991x over previous
import jax
import jax.numpy as jnp
from jax.experimental import pallas as pl
from jax.experimental.pallas import tpu as pltpu

_NORM = 100.0
_THRESH2 = 100.0


def _silu(x):
    h = 0.5 * x
    return h * jnp.tanh(h) + h


def _dot(a, b, prec=jax.lax.Precision.DEFAULT):
    return jax.lax.dot_general(
        a, b, (((a.ndim - 1,), (0,)), ((), ())),
        preferred_element_type=jnp.float32, precision=prec)


def _xattn_body(hl_ref, xl_ref, hp_ref, xp_ref, lm_ref, pmt_ref,
                Wa1l_ref, Wa1p_ref, wa1d_ref, ba1_ref,
                Wa2_ref, ba2_ref, wa3_ref, ba3_ref,
                Wv1p_ref, wv1d_ref, bv1_ref, Wv2_ref, bv2_ref,
                Wc1l_ref, Wc1p_ref, wc1d_ref, bc1_ref, wc2_ref, bc2_ref,
                h_out_ref, x_out_ref,
                hpA_ref, hpC_ref, hpV_ref):
    hl = hl_ref[0]
    xl = xl_ref[0]
    hp = hp_ref[0]
    xp = xp_ref[0]
    lm = lm_ref[0]
    pmt = pmt_ref[0]

    @pl.when(pl.program_id(1) == 0)
    def _():
        hpA_ref[:] = _dot(hp, Wa1p_ref[:])
        hpC_ref[:] = _dot(hp, Wc1p_ref[:])
        hpV_ref[:] = _dot(hp, Wv1p_ref[:]) + bv1_ref[:]

    xl2 = jnp.sum(xl * xl, axis=1, keepdims=True)
    xp2 = jnp.sum(xp * xp, axis=1, keepdims=True)
    d2 = jnp.maximum(xl2 + xp2.T - 2.0 * _dot(xl, xp.T, jax.lax.Precision.HIGHEST),
                     0.0)
    dist = jnp.sqrt(d2 + 1e-8)
    em = (d2 < _THRESH2).astype(jnp.float32)

    hlA = _dot(hl, Wa1l_ref[:]) + ba1_ref[:]
    hlC = _dot(hl, Wc1l_ref[:]) + bc1_ref[:]

    d2e = d2[:, :, None]

    a1 = _silu(hlA[:, None, :] + hpA_ref[:][None] + d2e * wa1d_ref[:][None])
    a2 = _silu(_dot(a1, Wa2_ref[:]) + ba2_ref[:][None])
    att = jax.nn.sigmoid(jnp.sum(a2 * wa3_ref[:][None], axis=2)
                         + ba3_ref[0, 0])
    ae = att * pmt * em

    v1 = _silu(hpV_ref[:][None] + d2e * wv1d_ref[:][None])
    t = jnp.sum(ae[:, :, None] * v1, axis=1)
    s = jnp.sum(ae, axis=1, keepdims=True)
    h_cross = (_dot(t, Wv2_ref[:]) + s * bv2_ref[:]) * (1.0 / _NORM) * lm

    c1 = _silu(hlC[:, None, :] + hpC_ref[:][None] + d2e * wc1d_ref[:][None])
    c = jnp.tanh(jnp.sum(c1 * wc2_ref[:][None], axis=2) + bc2_ref[0, 0])
    g = c * pmt * em / (dist + 1e-8)
    gs = jnp.sum(g, axis=1, keepdims=True)
    x_cross = (xl * gs - _dot(g, xp)) * (1.0 / _NORM) * lm

    h_out_ref[0] = h_cross
    x_out_ref[0] = x_cross


@jax.jit
def kernel(h_ligand, x_ligand, h_protein, x_protein, ligand_mask, protein_mask,
           W_a1, b_a1, W_a2, b_a2, W_a3, b_a3,
           W_v1, b_v1, W_v2, b_v2, W_c1, b_c1, W_c2, b_c2):
    bs, nl, lig_nf = h_ligand.shape
    npn, prot_nf = h_protein.shape[1], h_protein.shape[2]
    hid = W_a2.shape[0]
    f32 = jnp.float32
    TL = 24
    nt = nl // TL

    pmt = jnp.transpose(protein_mask, (0, 2, 1))

    args = (
        h_ligand, x_ligand, h_protein, x_protein, ligand_mask, pmt,
        W_a1[:lig_nf], W_a1[lig_nf:lig_nf + prot_nf], W_a1[lig_nf + prot_nf:],
        b_a1.reshape(1, -1),
        W_a2, b_a2.reshape(1, -1), W_a3.T, b_a3.reshape(1, 1),
        W_v1[:prot_nf], W_v1[prot_nf:], b_v1.reshape(1, -1),
        W_v2, b_v2.reshape(1, -1),
        W_c1[:lig_nf], W_c1[lig_nf:lig_nf + prot_nf], W_c1[lig_nf + prot_nf:],
        b_c1.reshape(1, -1), W_c2.T, b_c2.reshape(1, 1),
    )

    def lig_spec(a):
        shp = a.shape
        return pl.BlockSpec((1, TL) + shp[2:],
                            lambda b, t: (b, t) + (0,) * (len(shp) - 2))

    def batch_spec(a):
        shp = a.shape
        return pl.BlockSpec((1,) + shp[1:],
                            lambda b, t: (b,) + (0,) * (len(shp) - 1))

    def full_spec(a):
        shp = a.shape
        return pl.BlockSpec(shp, lambda b, t: (0,) * len(shp))

    in_specs = ([lig_spec(args[0]), lig_spec(args[1]),
                 batch_spec(args[2]), batch_spec(args[3]),
                 lig_spec(args[4]), batch_spec(args[5])]
                + [full_spec(a) for a in args[6:]])

    h_cross, x_cross = pl.pallas_call(
        _xattn_body,
        grid=(bs, nt),
        in_specs=in_specs,
        out_specs=[
            pl.BlockSpec((1, TL, lig_nf), lambda b, t: (b, t, 0)),
            pl.BlockSpec((1, TL, 3), lambda b, t: (b, t, 0)),
        ],
        out_shape=[
            jax.ShapeDtypeStruct((bs, nl, lig_nf), f32),
            jax.ShapeDtypeStruct((bs, nl, 3), f32),
        ],
        scratch_shapes=[
            pltpu.VMEM((npn, hid), f32),
            pltpu.VMEM((npn, hid), f32),
            pltpu.VMEM((npn, hid), f32),
        ],
    )(*args)
    return (h_cross, x_cross)
